# speedup vs baseline: 2.7706x; 2.7706x over previous
import functools

import jax
import jax.numpy as jnp
from jax import lax
from jax.experimental import pallas as pl
from jax.experimental.pallas import tpu as pltpu
from jax.experimental.pallas import tpu_sc as plsc

_N, _E, _D = 10000, 320000, 128
_NC, _NS = 2, 16
_NW = _NC * _NS
_K = 128
_CPW = 80
_EPW = _K * _CPW
_EPAD = _EPW * _NW
_RPAD = 10240
_RPT = _RPAD // _NS


def _mlp1(x, W1, b1):
    blk = 1000

    def body(x_ref, w_ref, b_ref, o_ref):
        h = jnp.dot(x_ref[...], w_ref[...], preferred_element_type=jnp.float32)
        o_ref[...] = jnp.maximum(h + b_ref[...], 0.0)

    return pl.pallas_call(
        body,
        grid=(_N // blk,),
        in_specs=[
            pl.BlockSpec((blk, _D), lambda i: (i, 0)),
            pl.BlockSpec((_D, _D), lambda i: (0, 0)),
            pl.BlockSpec((1, _D), lambda i: (0, 0)),
        ],
        out_specs=pl.BlockSpec((blk, _D), lambda i: (i, 0)),
        out_shape=jax.ShapeDtypeStruct((_N, _D), jnp.float32),
    )(x, W1, b1.reshape(1, _D))


def _sc_aggregate(h, row_p, col_p, zeros):
    mesh = plsc.VectorSubcoreMesh(core_axis_name="c", subcore_axis_name="s")

    @functools.partial(
        pl.kernel,
        mesh=mesh,
        out_type=jax.ShapeDtypeStruct((_NC, _RPAD, _D), jnp.float32),
        scratch_types=[
            pltpu.VMEM((_K,), jnp.int32),
            pltpu.VMEM((_K,), jnp.int32),
            pltpu.VMEM((_K, _D), jnp.float32),
            pltpu.VMEM_SHARED((_RPAD, _D), jnp.float32),
            pltpu.SemaphoreType.DMA,
        ],
    )
    def agg_kernel(h_hbm, row_hbm, col_hbm, z_hbm, out_hbm,
                   col_v, row_v, rows_v, acc_sh, sem):
        cid = lax.axis_index("c")
        sid = lax.axis_index("s")
        wid = sid * _NC + cid
        pltpu.sync_copy(z_hbm.at[pl.ds(sid * _RPT, _RPT)],
                        acc_sh.at[pl.ds(sid * _RPT, _RPT)])
        plsc.subcore_barrier()

        def body(c, carry):
            base = pl.multiple_of(wid * _EPW + c * _K, _K)
            pltpu.sync_copy(col_hbm.at[pl.ds(base, _K)], col_v)
            pltpu.sync_copy(row_hbm.at[pl.ds(base, _K)], row_v)
            pltpu.async_copy(h_hbm.at[col_v], rows_v, sem).wait()
            pltpu.sync_copy(rows_v, acc_sh.at[row_v], add=True)
            return carry

        lax.fori_loop(0, _CPW, body, 0)
        plsc.subcore_barrier()
        pltpu.sync_copy(acc_sh.at[pl.ds(sid * _RPT, _RPT)],
                        out_hbm.at[cid, pl.ds(sid * _RPT, _RPT)])

    return agg_kernel(h, row_p, col_p, zeros)


def _mlp2(a0, a1, W2, b2):
    blk = 1000

    def body(a0_ref, a1_ref, w_ref, b_ref, o_ref):
        agg = a0_ref[...] + a1_ref[...]
        out = jnp.dot(agg, w_ref[...], preferred_element_type=jnp.float32)
        out = out + b_ref[...]
        m = jnp.max(out, axis=1, keepdims=True)
        lse = jnp.log(jnp.sum(jnp.exp(out - m), axis=1, keepdims=True)) + m
        o_ref[...] = out - lse

    return pl.pallas_call(
        body,
        grid=(_N // blk,),
        in_specs=[
            pl.BlockSpec((blk, _D), lambda i: (i, 0)),
            pl.BlockSpec((blk, _D), lambda i: (i, 0)),
            pl.BlockSpec((_D, _D), lambda i: (0, 0)),
            pl.BlockSpec((1, _D), lambda i: (0, 0)),
        ],
        out_specs=pl.BlockSpec((blk, _D), lambda i: (i, 0)),
        out_shape=jax.ShapeDtypeStruct((_N, _D), jnp.float32),
    )(a0, a1, W2, b2.reshape(1, _D))


def kernel(x, adj_or_edge_index, W1, b1, W2, b2):
    row = adj_or_edge_index[0]
    col = adj_or_edge_index[1]
    pad = _EPAD - _E
    row_p = jnp.concatenate([row, jnp.full((pad,), _N, jnp.int32)])
    col_p = jnp.concatenate([col, jnp.zeros((pad,), jnp.int32)])
    h = _mlp1(x, W1, b1)
    zeros = jnp.zeros((_RPAD, _D), jnp.float32)
    agg = _sc_aggregate(h, row_p, col_p, zeros)
    return _mlp2(agg[0, :_N], agg[1, :_N], W2, b2)


# speedup vs baseline: 3.2116x; 1.1592x over previous
import functools

import jax
import jax.numpy as jnp
from jax import lax
from jax.experimental import pallas as pl
from jax.experimental.pallas import tpu as pltpu
from jax.experimental.pallas import tpu_sc as plsc

_N, _E, _D = 10000, 320000, 128
_NC, _NS = 2, 16
_NW = _NC * _NS
_K = 128
_CPW = 80
_EPW = _K * _CPW
_EPAD = _EPW * _NW
_RPAD = 10240
_RPT = _RPAD // _NS


def _mlp1(x, W1, b1):
    blk = 1000

    def body(x_ref, w_ref, b_ref, o_ref):
        h = jnp.dot(x_ref[...], w_ref[...], preferred_element_type=jnp.float32)
        o_ref[...] = jnp.maximum(h + b_ref[...], 0.0)

    return pl.pallas_call(
        body,
        grid=(_N // blk,),
        in_specs=[
            pl.BlockSpec((blk, _D), lambda i: (i, 0)),
            pl.BlockSpec((_D, _D), lambda i: (0, 0)),
            pl.BlockSpec((1, _D), lambda i: (0, 0)),
        ],
        out_specs=pl.BlockSpec((blk, _D), lambda i: (i, 0)),
        out_shape=jax.ShapeDtypeStruct((_N, _D), jnp.float32),
    )(x, W1, b1.reshape(1, _D))


def _sc_aggregate(h, row_p, col_p, zeros):
    mesh = plsc.VectorSubcoreMesh(core_axis_name="c", subcore_axis_name="s")

    @functools.partial(
        pl.kernel,
        mesh=mesh,
        out_type=jax.ShapeDtypeStruct((_NC, _RPAD, _D), jnp.float32),
        scratch_types=[
            pltpu.VMEM((_CPW, _K), jnp.int32),
            pltpu.VMEM((_K,), jnp.int32),
            pltpu.VMEM((_K,), jnp.int32),
            pltpu.VMEM((_K, _D), jnp.float32),
            pltpu.VMEM((_K, _D), jnp.float32),
            pltpu.VMEM_SHARED((_RPAD, _D), jnp.float32),
            pltpu.SemaphoreType.DMA,
            pltpu.SemaphoreType.DMA,
            pltpu.SemaphoreType.DMA,
            pltpu.SemaphoreType.DMA,
            pltpu.SemaphoreType.DMA,
            pltpu.SemaphoreType.DMA,
        ],
    )
    def agg_kernel(h_hbm, row_hbm, col_hbm, z_hbm, out_hbm,
                   colv, rowb0, rowb1, rows0, rows1, acc_sh,
                   gsem0, gsem1, ssem0, ssem1, rsem0, rsem1):
        cid = lax.axis_index("c")
        sid = lax.axis_index("s")
        wid = sid * _NC + cid
        pltpu.sync_copy(z_hbm.at[pl.ds(sid * _RPT, _RPT)],
                        acc_sh.at[pl.ds(sid * _RPT, _RPT)])
        pltpu.sync_copy(col_hbm.at[wid], colv)
        plsc.subcore_barrier()

        def rowload(c, rowb, sem):
            return pltpu.async_copy(row_hbm.at[wid, c], rowb, sem)

        def rowload_wait(c, rowb, sem):
            pltpu.make_async_copy(row_hbm.at[wid, c], rowb, sem).wait()

        def gather(c, rows, sem):
            return pltpu.async_copy(h_hbm.at[colv.at[c]], rows, sem)

        def gather_wait(c, rows, sem):
            pltpu.make_async_copy(h_hbm.at[colv.at[c]], rows, sem).wait()

        def scatter(rowb, rows, sem):
            return pltpu.async_copy(rows, acc_sh.at[rowb], sem, add=True)

        def scatter_wait(rowb, rows, sem):
            pltpu.make_async_copy(rows, acc_sh.at[rowb], sem).wait()

        rowload(0, rowb0, rsem0)
        rowload(1, rowb1, rsem1)
        gather(0, rows0, gsem0)

        def body(i, carry):
            c0 = 2 * i
            c1 = c0 + 1
            gather_wait(c0, rows0, gsem0)
            gather(c1, rows1, gsem1)
            rowload_wait(c0, rowb0, rsem0)
            scatter(rowb0, rows0, ssem0)
            gather_wait(c1, rows1, gsem1)
            scatter_wait(rowb0, rows0, ssem0)

            @pl.when(i < _CPW // 2 - 1)
            def _():
                rowload(c0 + 2, rowb0, rsem0)
                gather(c0 + 2, rows0, gsem0)

            rowload_wait(c1, rowb1, rsem1)
            scatter(rowb1, rows1, ssem1)
            scatter_wait(rowb1, rows1, ssem1)

            @pl.when(i < _CPW // 2 - 1)
            def _():
                rowload(c1 + 2, rowb1, rsem1)

            return carry

        lax.fori_loop(0, _CPW // 2, body, 0)
        plsc.subcore_barrier()
        pltpu.sync_copy(acc_sh.at[pl.ds(sid * _RPT, _RPT)],
                        out_hbm.at[cid, pl.ds(sid * _RPT, _RPT)])

    return agg_kernel(h, row_p, col_p, zeros)


def _mlp2(a0, a1, W2, b2):
    blk = 1000

    def body(a0_ref, a1_ref, w_ref, b_ref, o_ref):
        agg = a0_ref[...] + a1_ref[...]
        out = jnp.dot(agg, w_ref[...], preferred_element_type=jnp.float32)
        out = out + b_ref[...]
        m = jnp.max(out, axis=1, keepdims=True)
        lse = jnp.log(jnp.sum(jnp.exp(out - m), axis=1, keepdims=True)) + m
        o_ref[...] = out - lse

    return pl.pallas_call(
        body,
        grid=(_N // blk,),
        in_specs=[
            pl.BlockSpec((blk, _D), lambda i: (i, 0)),
            pl.BlockSpec((blk, _D), lambda i: (i, 0)),
            pl.BlockSpec((_D, _D), lambda i: (0, 0)),
            pl.BlockSpec((1, _D), lambda i: (0, 0)),
        ],
        out_specs=pl.BlockSpec((blk, _D), lambda i: (i, 0)),
        out_shape=jax.ShapeDtypeStruct((_N, _D), jnp.float32),
    )(a0, a1, W2, b2.reshape(1, _D))


def kernel(x, adj_or_edge_index, W1, b1, W2, b2):
    row = adj_or_edge_index[0]
    col = adj_or_edge_index[1]
    pad = _EPAD - _E
    row_p = jnp.concatenate([row, jnp.full((pad,), _N, jnp.int32)])
    col_p = jnp.concatenate([col, jnp.zeros((pad,), jnp.int32)])
    row_p = row_p.reshape(_NW, _CPW, _K)
    col_p = col_p.reshape(_NW, _CPW, _K)
    h = _mlp1(x, W1, b1)
    zeros = jnp.zeros((_RPAD, _D), jnp.float32)
    agg = _sc_aggregate(h, row_p, col_p, zeros)
    return _mlp2(agg[0, :_N], agg[1, :_N], W2, b2)
